# baseline (speedup 1.0000x reference)
import jax
import jax.numpy as jnp
from jax.experimental import pallas as pl
from jax.experimental.pallas import tpu as pltpu

_BSZ, _SEQ, _DIN, _DOUT, _E, _K, _R = 4, 2048, 768, 768, 64, 8, 16


def _moelora_body(idx_ref, attn_ref, x_ref, wt_ref, b_ref, ap_hbm, bp_hbm,
                  bias_ref, out_ref, a_sc, b_sc, sems):
    bi = pl.program_id(0)
    for k in range(_K):
        e = idx_ref[bi, k]
        pltpu.make_async_copy(
            ap_hbm.at[pl.ds(e, 1)], a_sc.at[pl.ds(k, 1)], sems.at[k]).start()
        pltpu.make_async_copy(
            bp_hbm.at[pl.ds(e, 1)], b_sc.at[pl.ds(k, 1)], sems.at[_K + k]).start()
    bias_acc = b_ref[:]
    for k in range(_K):
        w = attn_ref[bi, k]
        e = idx_ref[bi, k]
        bias_acc = bias_acc + w * bias_ref[pl.ds(e, 1), :]
    for k in range(_K):
        e = idx_ref[bi, k]
        pltpu.make_async_copy(
            ap_hbm.at[pl.ds(e, 1)], a_sc.at[pl.ds(k, 1)], sems.at[k]).wait()
        pltpu.make_async_copy(
            bp_hbm.at[pl.ds(e, 1)], b_sc.at[pl.ds(k, 1)], sems.at[_K + k]).wait()
    a_parts = []
    b_parts = []
    for k in range(_K):
        w = attn_ref[bi, k]
        a_parts.append(a_sc[k].astype(jnp.bfloat16))
        b_parts.append((b_sc[k] * w).astype(jnp.bfloat16))
    acat = jnp.concatenate(a_parts, axis=1)
    bcat = jnp.concatenate(b_parts, axis=0)
    delta_t = jax.lax.dot_general(
        bcat, acat, (((0,), (1,)), ((), ())),
        preferred_element_type=jnp.float32)
    m_t = (wt_ref[:] + delta_t).astype(jnp.bfloat16)
    out_ref[0] = jax.lax.dot_general(
        x_ref[0].astype(jnp.bfloat16), m_t,
        (((1,), (1,)), ((), ())),
        preferred_element_type=jnp.float32) + bias_acc


@jax.jit
def _run(x, attn, idx, w, b2, ap, bp, bias_pool):
    return pl.pallas_call(
        _moelora_body,
        grid=(_BSZ,),
        in_specs=[
            pl.BlockSpec(memory_space=pltpu.SMEM),
            pl.BlockSpec(memory_space=pltpu.SMEM),
            pl.BlockSpec((1, _SEQ, _DIN), lambda i: (i, 0, 0)),
            pl.BlockSpec((_DOUT, _DIN), lambda i: (0, 0)),
            pl.BlockSpec((1, _DOUT), lambda i: (0, 0)),
            pl.BlockSpec(memory_space=pl.ANY),
            pl.BlockSpec(memory_space=pl.ANY),
            pl.BlockSpec((_E, _DOUT), lambda i: (0, 0)),
        ],
        out_specs=pl.BlockSpec((1, _SEQ, _DOUT), lambda i: (i, 0, 0)),
        out_shape=jax.ShapeDtypeStruct((_BSZ, _SEQ, _DOUT), jnp.float32),
        scratch_shapes=[
            pltpu.VMEM((_K, _DIN, _R), jnp.float32),
            pltpu.VMEM((_K, _R, _DOUT), jnp.float32),
            pltpu.SemaphoreType.DMA((2 * _K,)),
        ],
    )(idx, attn, x, w, b2, ap, bp, bias_pool)


def kernel(x, topk_attn, topk_idx, W, b, A_pool, B_pool, bias_pool):
    b2 = b.reshape(1, _DOUT)
    idx = topk_idx.astype(jnp.int32)
    return _run(x, topk_attn, idx, W, b2, A_pool, B_pool, bias_pool)
